# baseline (speedup 1.0000x reference)
import functools
import math

import jax
import jax.numpy as jnp
from jax.experimental import pallas as pl
from jax.experimental.pallas import tpu as pltpu
from jax.experimental.pallas import tpu_sc as plsc

B = 16
N = 32768
D = 256
H1 = 512
H2 = 256
C = 50

_NC = 2
_NS = 16
_NW = _NC * _NS
_RPW = N // _NW
_CH = 128
_NCH = _RPW // _CH
_DL = D // 16
_NCHUNK = N // _CH
_OUTROWS = _NCHUNK + _NW


def _sc_segsum_body(feat_hbm, offs_hbm, out_hbm,
                    offs_v, buf0, buf1, snap, off_s, sem0, sem1):
    cid = jax.lax.axis_index("c")
    sid = jax.lax.axis_index("s")
    wid = sid * _NC + cid
    lo = wid * _RPW

    pltpu.async_copy(feat_hbm.at[pl.ds(lo, _CH)], buf0, sem0)
    pltpu.async_copy(feat_hbm.at[pl.ds(lo + _CH, _CH)], buf1, sem1)

    pltpu.sync_copy(offs_hbm.at[pl.ds(0, 16)], offs_v)
    offv = offs_v[pl.ds(0, 16)]
    for i in range(16):
        off_s[i] = offv[i]

    zero = jnp.zeros((16,), jnp.float32)

    def sum_rows(buf, nrows, unroll):
        def body(r, regs):
            return tuple(regs[dd] + buf[r, pl.ds(dd * 16, 16)]
                         for dd in range(_DL))
        return plsc.parallel_loop(
            0, nrows, 1, unroll=unroll,
            carry=tuple(zero for _ in range(_DL)))(body)

    def store_row(row, regs):
        for dd in range(_DL):
            snap[row, pl.ds(dd * 16, 16)] = regs[dd]

    def loop_body(j, carry):
        c0 = j * 2
        for half, (buf, sem) in enumerate(((buf0, sem0), (buf1, sem1))):
            c = c0 + half
            pltpu.make_async_copy(feat_hbm.at[pl.ds(0, _CH)], buf, sem).wait()
            store_row(c, sum_rows(buf, _CH, 4))

            @pl.when(c + 2 < _NCH)
            def _():
                pltpu.async_copy(
                    feat_hbm.at[pl.ds(lo + (c + 2) * _CH, _CH)], buf, sem)
        return carry

    jax.lax.fori_loop(0, _NCH // 2, loop_body, 0)

    b = off_s[jnp.minimum(wid, 14) + 1]
    nrows = jax.lax.rem(b, _CH)
    start = pl.multiple_of(b - nrows, _CH)
    pltpu.sync_copy(feat_hbm.at[pl.ds(start, _CH)], buf0)
    store_row(_NCH, sum_rows(buf0, nrows, 1))

    pltpu.sync_copy(snap.at[pl.ds(0, _NCH)],
                    out_hbm.at[pl.ds(wid * _NCH, _NCH)])
    pltpu.sync_copy(snap.at[pl.ds(_NCH, 1)],
                    out_hbm.at[pl.ds(_NCHUNK + wid, 1)])


_sc_segsum = functools.partial(
    pl.kernel,
    out_type=jax.ShapeDtypeStruct((_OUTROWS, D), jnp.float32),
    mesh=plsc.VectorSubcoreMesh(core_axis_name="c", subcore_axis_name="s"),
    scratch_types=[
        pltpu.VMEM((16,), jnp.int32),
        pltpu.VMEM((_CH, D), jnp.float32),
        pltpu.VMEM((_CH, D), jnp.float32),
        pltpu.VMEM((_NCH + 1, D), jnp.float32),
        pltpu.SMEM((16,), jnp.int32),
        pltpu.SemaphoreType.DMA,
        pltpu.SemaphoreType.DMA,
    ],
)(_sc_segsum_body)


def _chunk_coeffs(off):
    c = jnp.arange(_NCHUNK, dtype=jnp.int32)
    Pchunk = (c[None, :] < (off[:, None] // _CH)).astype(jnp.float32)
    Ppart = jnp.zeros((B + 1, _NW), jnp.float32)
    Ppart = Ppart.at[jnp.arange(1, 16), jnp.arange(0, 15)].set(1.0)
    P = jnp.concatenate([Pchunk, Ppart], axis=1)
    return P[1:] - P[:-1]


def _erf(x):
    a1, a2, a3, a4, a5 = (0.254829592, -0.284496736, 1.421413741,
                          -1.453152027, 1.061405429)
    p = 0.3275911
    ax = jnp.abs(x)
    t = 1.0 / (1.0 + p * ax)
    poly = t * (a1 + t * (a2 + t * (a3 + t * (a4 + t * a5))))
    y = 1.0 - poly * jnp.exp(-ax * ax)
    return jnp.sign(x) * y


def _gelu(x):
    return 0.5 * x * (1.0 + _erf(x * (1.0 / math.sqrt(2.0))))


def _layernorm(x, g, b, eps=1e-5):
    m = jnp.mean(x, axis=-1, keepdims=True)
    v = jnp.mean((x - m) ** 2, axis=-1, keepdims=True)
    return (x - m) * jax.lax.rsqrt(v + eps) * g + b


def _mlp_body(snaps_ref, coef_ref, counts_ref, W1_ref, b1_ref, g1_ref,
              be1_ref, W2_ref, b2_ref, g2_ref, be2_ref, W3_ref, b3_ref,
              out_ref):
    sums = jnp.dot(coef_ref[...], snaps_ref[...],
                   preferred_element_type=jnp.float32)
    means = sums / jnp.maximum(counts_ref[...], 1.0)
    h = jnp.dot(means, W1_ref[...], preferred_element_type=jnp.float32)
    h = h + b1_ref[...]
    h = _layernorm(h, g1_ref[...], be1_ref[...])
    h = _gelu(h)
    h = jnp.dot(h, W2_ref[...], preferred_element_type=jnp.float32)
    h = h + b2_ref[...]
    h = _layernorm(h, g2_ref[...], be2_ref[...])
    h = _gelu(h)
    out = jnp.dot(h, W3_ref[...], preferred_element_type=jnp.float32)
    out_ref[...] = out + b3_ref[...]


@jax.jit
def kernel(feat, offsets, W1, b1, g1, be1, W2, b2, g2, be2, W3, b3):
    off = offsets.astype(jnp.int32)
    counts = (off[1:] - off[:-1]).reshape(B, 1).astype(jnp.float32)
    coef = _chunk_coeffs(off)

    snaps = _sc_segsum(feat, off)

    out = pl.pallas_call(
        _mlp_body,
        in_specs=[
            pl.BlockSpec((_OUTROWS, D), lambda: (0, 0)),
            pl.BlockSpec((B, _OUTROWS), lambda: (0, 0)),
            pl.BlockSpec((B, 1), lambda: (0, 0)),
            pl.BlockSpec((D, H1), lambda: (0, 0)),
            pl.BlockSpec((1, H1), lambda: (0, 0)),
            pl.BlockSpec((1, H1), lambda: (0, 0)),
            pl.BlockSpec((1, H1), lambda: (0, 0)),
            pl.BlockSpec((H1, H2), lambda: (0, 0)),
            pl.BlockSpec((1, H2), lambda: (0, 0)),
            pl.BlockSpec((1, H2), lambda: (0, 0)),
            pl.BlockSpec((1, H2), lambda: (0, 0)),
            pl.BlockSpec((H2, C), lambda: (0, 0)),
            pl.BlockSpec((1, C), lambda: (0, 0)),
        ],
        out_specs=pl.BlockSpec((B, C), lambda: (0, 0)),
        out_shape=jax.ShapeDtypeStruct((B, C), jnp.float32),
    )(snaps, coef, counts, W1, b1.reshape(1, H1), g1.reshape(1, H1),
      be1.reshape(1, H1), W2, b2.reshape(1, H2), g2.reshape(1, H2),
      be2.reshape(1, H2), W3, b3.reshape(1, C))
    return out


# speedup vs baseline: 1.0906x; 1.0906x over previous
import functools
import math

import jax
import jax.numpy as jnp
from jax.experimental import pallas as pl
from jax.experimental.pallas import tpu as pltpu
from jax.experimental.pallas import tpu_sc as plsc

B = 16
N = 32768
D = 256
H1 = 512
H2 = 256
C = 50

_NC = 2
_NS = 16
_NW = _NC * _NS
_CH = 128
_DL = D // 16
_NCHUNK = N // _CH
_TCG = 8
_RPG = N // _TCG
_CPG = _NCHUNK // _TCG


def _sc_partials_body(feat_hbm, offs_hbm, out_hbm,
                      offs_v, buf, row, off_s, sem):
    cid = jax.lax.axis_index("c")
    sid = jax.lax.axis_index("s")
    wid = sid * _NC + cid

    pltpu.sync_copy(offs_hbm.at[pl.ds(0, 16)], offs_v)
    offv = offs_v[pl.ds(0, 16)]
    for i in range(16):
        off_s[i] = offv[i]

    b = off_s[jnp.minimum(wid, 14) + 1]
    nrows = jax.lax.rem(b, _CH)
    start = pl.multiple_of(b - nrows, _CH)
    pltpu.sync_copy(feat_hbm.at[pl.ds(start, _CH)], buf)

    zero = jnp.zeros((16,), jnp.float32)

    def body(r, regs):
        return tuple(regs[dd] + buf[r, pl.ds(dd * 16, 16)]
                     for dd in range(_DL))
    regs = plsc.parallel_loop(
        0, nrows, 1, carry=tuple(zero for _ in range(_DL)))(body)

    for dd in range(_DL):
        row[0, pl.ds(dd * 16, 16)] = regs[dd]
    pltpu.sync_copy(row, out_hbm.at[pl.ds(wid, 1)])


_sc_partials = functools.partial(
    pl.kernel,
    out_type=jax.ShapeDtypeStruct((_NW, D), jnp.float32),
    mesh=plsc.VectorSubcoreMesh(core_axis_name="c", subcore_axis_name="s"),
    scratch_types=[
        pltpu.VMEM((16,), jnp.int32),
        pltpu.VMEM((_CH, D), jnp.float32),
        pltpu.VMEM((1, D), jnp.float32),
        pltpu.SMEM((16,), jnp.int32),
        pltpu.SemaphoreType.DMA,
    ],
)(_sc_partials_body)


def _chunksum_body(x_ref, o_ref):
    r = jax.lax.broadcasted_iota(jnp.int32, (_CPG, _RPG), 1)
    k = jax.lax.broadcasted_iota(jnp.int32, (_CPG, _RPG), 0)
    sel = (r // _CH == k).astype(jnp.float32)
    o_ref[...] = jnp.dot(sel, x_ref[...], preferred_element_type=jnp.float32)


def _coeffs(off):
    c = jnp.arange(_NCHUNK, dtype=jnp.int32)
    Pchunk = (c[None, :] < (off[:, None] // _CH)).astype(jnp.float32)
    Ppart = jnp.zeros((B + 1, _NW), jnp.float32)
    Ppart = Ppart.at[jnp.arange(1, 16), jnp.arange(0, 15)].set(1.0)
    return Pchunk[1:] - Pchunk[:-1], Ppart[1:] - Ppart[:-1]


def _erf(x):
    a1, a2, a3, a4, a5 = (0.254829592, -0.284496736, 1.421413741,
                          -1.453152027, 1.061405429)
    p = 0.3275911
    ax = jnp.abs(x)
    t = 1.0 / (1.0 + p * ax)
    poly = t * (a1 + t * (a2 + t * (a3 + t * (a4 + t * a5))))
    y = 1.0 - poly * jnp.exp(-ax * ax)
    return jnp.sign(x) * y


def _gelu(x):
    return 0.5 * x * (1.0 + _erf(x * (1.0 / math.sqrt(2.0))))


def _layernorm(x, g, b, eps=1e-5):
    m = jnp.mean(x, axis=-1, keepdims=True)
    v = jnp.mean((x - m) ** 2, axis=-1, keepdims=True)
    return (x - m) * jax.lax.rsqrt(v + eps) * g + b


def _mlp_body(chunks_ref, parts_ref, coefc_ref, coefp_ref, counts_ref,
              W1_ref, b1_ref, g1_ref, be1_ref, W2_ref, b2_ref, g2_ref,
              be2_ref, W3_ref, b3_ref, out_ref):
    sums = (jnp.dot(coefc_ref[...], chunks_ref[...],
                    preferred_element_type=jnp.float32)
            + jnp.dot(coefp_ref[...], parts_ref[...],
                      preferred_element_type=jnp.float32))
    means = sums / jnp.maximum(counts_ref[...], 1.0)
    h = jnp.dot(means, W1_ref[...], preferred_element_type=jnp.float32)
    h = h + b1_ref[...]
    h = _layernorm(h, g1_ref[...], be1_ref[...])
    h = _gelu(h)
    h = jnp.dot(h, W2_ref[...], preferred_element_type=jnp.float32)
    h = h + b2_ref[...]
    h = _layernorm(h, g2_ref[...], be2_ref[...])
    h = _gelu(h)
    out = jnp.dot(h, W3_ref[...], preferred_element_type=jnp.float32)
    out_ref[...] = out + b3_ref[...]


@jax.jit
def kernel(feat, offsets, W1, b1, g1, be1, W2, b2, g2, be2, W3, b3):
    off = offsets.astype(jnp.int32)
    counts = (off[1:] - off[:-1]).reshape(B, 1).astype(jnp.float32)
    coefc, coefp = _coeffs(off)

    parts = _sc_partials(feat, off)

    chunks = pl.pallas_call(
        _chunksum_body,
        grid=(_TCG,),
        in_specs=[pl.BlockSpec((_RPG, D), lambda i: (i, 0))],
        out_specs=pl.BlockSpec((_CPG, D), lambda i: (i, 0)),
        out_shape=jax.ShapeDtypeStruct((_NCHUNK, D), jnp.float32),
    )(feat)

    out = pl.pallas_call(
        _mlp_body,
        in_specs=[
            pl.BlockSpec((_NCHUNK, D), lambda: (0, 0)),
            pl.BlockSpec((_NW, D), lambda: (0, 0)),
            pl.BlockSpec((B, _NCHUNK), lambda: (0, 0)),
            pl.BlockSpec((B, _NW), lambda: (0, 0)),
            pl.BlockSpec((B, 1), lambda: (0, 0)),
            pl.BlockSpec((D, H1), lambda: (0, 0)),
            pl.BlockSpec((1, H1), lambda: (0, 0)),
            pl.BlockSpec((1, H1), lambda: (0, 0)),
            pl.BlockSpec((1, H1), lambda: (0, 0)),
            pl.BlockSpec((H1, H2), lambda: (0, 0)),
            pl.BlockSpec((1, H2), lambda: (0, 0)),
            pl.BlockSpec((1, H2), lambda: (0, 0)),
            pl.BlockSpec((1, H2), lambda: (0, 0)),
            pl.BlockSpec((H2, C), lambda: (0, 0)),
            pl.BlockSpec((1, C), lambda: (0, 0)),
        ],
        out_specs=pl.BlockSpec((B, C), lambda: (0, 0)),
        out_shape=jax.ShapeDtypeStruct((B, C), jnp.float32),
    )(chunks, parts, coefc, coefp, counts, W1, b1.reshape(1, H1),
      g1.reshape(1, H1), be1.reshape(1, H1), W2, b2.reshape(1, H2),
      g2.reshape(1, H2), be2.reshape(1, H2), W3, b3.reshape(1, C))
    return out
